# baseline (speedup 1.0000x reference)
import functools

import jax
import jax.numpy as jnp
from jax import lax
from jax.experimental import pallas as pl
from jax.experimental.pallas import tpu as pltpu
from jax.experimental.pallas import tpu_sc as plsc

N = 10000
E = 320000
IN_FEATS = 128
HIDDEN = 32
HEADS = 4
NUM_CLASSES = 16

CHUNK = 3200
GPC = CHUNK // 16

_SC_MESH = plsc.VectorSubcoreMesh(core_axis_name="c", subcore_axis_name="s")
_SC_PARAMS = pltpu.CompilerParams(needs_layout_passes=False)


def _tc1_body(x_ref, w1_ref, a1_ref, featT_ref, eler_ref):
    featT = lax.dot_general(w1_ref[...], x_ref[...],
                            (((0,), (1,)), ((), ())),
                            preferred_element_type=jnp.float32)
    featT_ref[...] = featT
    eler_ref[...] = jnp.dot(a1_ref[...], featT,
                            preferred_element_type=jnp.float32)


def _tc2_body(xaggT_ref, b1_ref, w2_ref, a2_ref, featT2_ref, eler2_ref):
    x2 = jnp.maximum(xaggT_ref[...] + b1_ref[...], 0.0)
    featT2 = lax.dot_general(w2_ref[...], x2,
                             (((0,), (0,)), ((), ())),
                             preferred_element_type=jnp.float32)
    featT2_ref[...] = featT2
    eler2_ref[...] = jnp.dot(a2_ref[...], featT2,
                             preferred_element_type=jnp.float32)


def _tc3_body(accP_ref, denP_ref, b2_ref, out_ref):
    acc = accP_ref[0] + accP_ref[1]
    den = denP_ref[0:1, :] + denP_ref[1:2, :]
    out_ref[...] = acc / (den + 1e-9) + b2_ref[...]


def _sc1_body(featT_hbm, eler_hbm, eidx_hbm, xagg_hbm,
              feat_v, acc_v, el_v, er_v, den_v, ibuf, sem0, sem1):
    cid = lax.axis_index("c")
    sid = lax.axis_index("s")
    t = sid * 2 + cid
    h = t // 8

    pltpu.sync_copy(featT_hbm.at[pl.ds(4 * t, 4)], feat_v)
    pltpu.sync_copy(eler_hbm.at[pl.ds(h, 1)], el_v)
    pltpu.sync_copy(eler_hbm.at[pl.ds(4 + h, 1)], er_v)

    z = jnp.zeros((16,), jnp.float32)

    @plsc.parallel_loop(0, N // 16, 1, unroll=8)
    def zero_body(j):
        den_v[pl.ds(j * 16, 16)] = z
        for c in range(4):
            acc_v[c, pl.ds(j * 16, 16)] = z

    nchunks = E // CHUNK
    sems = (sem0, sem1)
    pltpu.async_copy(eidx_hbm.at[:, pl.ds(0, CHUNK)], ibuf.at[0], sem0)
    pltpu.async_copy(eidx_hbm.at[:, pl.ds(CHUNK, CHUNK)], ibuf.at[1], sem1)

    z16 = jnp.zeros((16,), jnp.int32)
    cvecs = [jnp.full((16,), c, jnp.int32) for c in range(4)]

    def grp(b):
        @plsc.parallel_loop(0, GPC, 1, unroll=8)
        def inner(g):
            s16 = ibuf[b, 0, pl.ds(g * 16, 16)]
            d16 = ibuf[b, 1, pl.ds(g * 16, 16)]
            a = plsc.load_gather(el_v, [z16, s16])
            r = plsc.load_gather(er_v, [z16, d16])
            e = a + r
            w = jnp.exp(jnp.maximum(e, 0.2 * e))
            plsc.addupdate_scatter(den_v, [d16], w)
            for c in range(4):
                v = plsc.load_gather(feat_v, [cvecs[c], s16])
                plsc.addupdate_scatter(acc_v, [cvecs[c], d16], v * w)

    def chunk_body(k, _):
        for b in (0, 1):
            ch = 2 * k + b
            pltpu.make_async_copy(
                eidx_hbm.at[:, pl.ds(0, CHUNK)], ibuf.at[b], sems[b]).wait()
            grp(b)
            nxt = jnp.minimum((ch + 2) * CHUNK, E - CHUNK)
            pltpu.async_copy(eidx_hbm.at[:, pl.ds(nxt, CHUNK)],
                             ibuf.at[b], sems[b])
        return 0

    lax.fori_loop(0, nchunks // 2, chunk_body, 0)
    pltpu.make_async_copy(eidx_hbm.at[:, pl.ds(0, CHUNK)], ibuf.at[0], sem0).wait()
    pltpu.make_async_copy(eidx_hbm.at[:, pl.ds(0, CHUNK)], ibuf.at[1], sem1).wait()

    one = jnp.full((16,), 1.0, jnp.float32)
    eps = jnp.full((16,), 1e-9, jnp.float32)

    @plsc.parallel_loop(0, N // 16, 1, unroll=8)
    def div_body(j):
        inv = one / (den_v[pl.ds(j * 16, 16)] + eps)
        for c in range(4):
            acc_v[c, pl.ds(j * 16, 16)] = acc_v[c, pl.ds(j * 16, 16)] * inv
    pltpu.sync_copy(acc_v, xagg_hbm.at[pl.ds(4 * t, 4)])


def _sc2_body(featT2_hbm, eler2_hbm, eidx_hbm, accP_hbm, denP_hbm,
              feat_v, acc_v, el_v, er_v, den_v, ibuf, sem0, sem1):
    cid = lax.axis_index("c")
    sid = lax.axis_index("s")
    col = sid
    shard = cid
    base_e = shard * (E // 2)

    pltpu.sync_copy(featT2_hbm.at[pl.ds(col, 1)], feat_v)
    pltpu.sync_copy(eler2_hbm.at[pl.ds(0, 1)], el_v)
    pltpu.sync_copy(eler2_hbm.at[pl.ds(1, 1)], er_v)

    z = jnp.zeros((16,), jnp.float32)

    @plsc.parallel_loop(0, N // 16, 1, unroll=8)
    def zero_body(j):
        den_v[pl.ds(j * 16, 16)] = z
        acc_v[pl.ds(j * 16, 16)] = z

    nchunks = (E // 2) // CHUNK
    sems = (sem0, sem1)
    pltpu.async_copy(eidx_hbm.at[:, pl.ds(base_e, CHUNK)], ibuf.at[0], sem0)
    pltpu.async_copy(eidx_hbm.at[:, pl.ds(base_e + CHUNK, CHUNK)], ibuf.at[1], sem1)

    is_den_tile = col == 0
    z16 = jnp.zeros((16,), jnp.int32)

    def grp(b):
        @plsc.parallel_loop(0, GPC, 1, unroll=8)
        def inner(g):
            s16 = ibuf[b, 0, pl.ds(g * 16, 16)]
            d16 = ibuf[b, 1, pl.ds(g * 16, 16)]
            a = plsc.load_gather(el_v, [z16, s16])
            r = plsc.load_gather(er_v, [z16, d16])
            e = a + r
            w = jnp.exp(jnp.maximum(e, 0.2 * e))

            @pl.when(is_den_tile)
            def _():
                plsc.addupdate_scatter(den_v, [d16], w)

            v = plsc.load_gather(feat_v, [z16, s16])
            plsc.addupdate_scatter(acc_v, [d16], v * w)

    def chunk_body(k, _):
        for b in (0, 1):
            ch = 2 * k + b
            pltpu.make_async_copy(
                eidx_hbm.at[:, pl.ds(0, CHUNK)], ibuf.at[b], sems[b]).wait()
            grp(b)
            nxt = base_e + jnp.minimum((ch + 2) * CHUNK, E // 2 - CHUNK)
            pltpu.async_copy(eidx_hbm.at[:, pl.ds(nxt, CHUNK)],
                             ibuf.at[b], sems[b])
        return 0

    lax.fori_loop(0, nchunks // 2, chunk_body, 0)
    pltpu.make_async_copy(eidx_hbm.at[:, pl.ds(0, CHUNK)], ibuf.at[0], sem0).wait()
    pltpu.make_async_copy(eidx_hbm.at[:, pl.ds(0, CHUNK)], ibuf.at[1], sem1).wait()

    w2id = shard * 16 + col
    pltpu.sync_copy(acc_v, accP_hbm.at[w2id])

    @pl.when(is_den_tile)
    def _():
        pltpu.sync_copy(den_v, denP_hbm.at[shard])


_sc1 = functools.partial(
    pl.kernel,
    out_type=jax.ShapeDtypeStruct((IN_FEATS, N), jnp.float32),
    mesh=_SC_MESH,
    compiler_params=_SC_PARAMS,
    scratch_types=[
        pltpu.VMEM((4, N), jnp.float32),
        pltpu.VMEM((4, N), jnp.float32),
        pltpu.VMEM((1, N), jnp.float32),
        pltpu.VMEM((1, N), jnp.float32),
        pltpu.VMEM((N,), jnp.float32),
        pltpu.VMEM((2, 2, CHUNK), jnp.int32),
        pltpu.SemaphoreType.DMA,
        pltpu.SemaphoreType.DMA,
    ],
)(_sc1_body)

_sc2 = functools.partial(
    pl.kernel,
    out_type=(jax.ShapeDtypeStruct((32, N), jnp.float32),
              jax.ShapeDtypeStruct((2, N), jnp.float32)),
    mesh=_SC_MESH,
    compiler_params=_SC_PARAMS,
    scratch_types=[
        pltpu.VMEM((1, N), jnp.float32),
        pltpu.VMEM((N,), jnp.float32),
        pltpu.VMEM((1, N), jnp.float32),
        pltpu.VMEM((1, N), jnp.float32),
        pltpu.VMEM((N,), jnp.float32),
        pltpu.VMEM((2, 2, CHUNK), jnp.int32),
        pltpu.SemaphoreType.DMA,
        pltpu.SemaphoreType.DMA,
    ],
)(_sc2_body)

_tc1 = pl.pallas_call(
    _tc1_body,
    out_shape=(jax.ShapeDtypeStruct((IN_FEATS, N), jnp.float32),
               jax.ShapeDtypeStruct((2 * HEADS, N), jnp.float32)),
)

_tc2 = pl.pallas_call(
    _tc2_body,
    out_shape=(jax.ShapeDtypeStruct((NUM_CLASSES, N), jnp.float32),
               jax.ShapeDtypeStruct((8, N), jnp.float32)),
)

_tc3 = pl.pallas_call(
    _tc3_body,
    out_shape=jax.ShapeDtypeStruct((NUM_CLASSES, N), jnp.float32),
)


def kernel(features, edge_index, W1, al1, ar1, b1, W2, al2, ar2, b2):
    eidx = edge_index.astype(jnp.int32)

    a1 = jnp.zeros((2 * HEADS, HEADS * HIDDEN), jnp.float32)
    for hh in range(HEADS):
        a1 = a1.at[hh, hh * HIDDEN:(hh + 1) * HIDDEN].set(al1[hh])
        a1 = a1.at[HEADS + hh, hh * HIDDEN:(hh + 1) * HIDDEN].set(ar1[hh])
    a2 = jnp.zeros((8, NUM_CLASSES), jnp.float32)
    a2 = a2.at[0].set(al2[0])
    a2 = a2.at[1].set(ar2[0])

    featT1, eler1 = _tc1(features, W1, a1)
    xaggT = _sc1(featT1, eler1, eidx)
    featT2, eler2 = _tc2(xaggT, b1.reshape(IN_FEATS, 1), W2, a2)
    accP, denP = _sc2(featT2, eler2, eidx)
    accP = accP.reshape(2, NUM_CLASSES, N)
    outT = _tc3(accP, denP, b2.reshape(NUM_CLASSES, 1))
    return outT.T


# speedup vs baseline: 1.4268x; 1.4268x over previous
import functools

import jax
import jax.numpy as jnp
from jax import lax
from jax.experimental import pallas as pl
from jax.experimental.pallas import tpu as pltpu
from jax.experimental.pallas import tpu_sc as plsc

N = 10000
E = 320000
IN_FEATS = 128
HIDDEN = 32
HEADS = 4
NUM_CLASSES = 16

CHUNK = 3200
GPC = CHUNK // 16

_SC_MESH = plsc.VectorSubcoreMesh(core_axis_name="c", subcore_axis_name="s")
_SC_PARAMS = pltpu.CompilerParams(needs_layout_passes=False)


def _tc1_body(x_ref, w1_ref, a1_ref, featT_ref, eler_ref):
    featT = lax.dot_general(w1_ref[...], x_ref[...],
                            (((0,), (1,)), ((), ())),
                            preferred_element_type=jnp.float32)
    featT_ref[...] = featT
    eler_ref[...] = jnp.dot(a1_ref[...], featT,
                            preferred_element_type=jnp.float32)


def _tc2_body(xaggT_ref, b1_ref, w2_ref, a2_ref, featT2_ref, eler2_ref):
    x2 = jnp.maximum(xaggT_ref[...] + b1_ref[...], 0.0)
    featT2 = lax.dot_general(w2_ref[...], x2,
                             (((0,), (0,)), ((), ())),
                             preferred_element_type=jnp.float32)
    featT2_ref[...] = featT2
    eler2_ref[...] = jnp.dot(a2_ref[...], featT2,
                             preferred_element_type=jnp.float32)


def _tc3_body(accP_ref, denP_ref, b2_ref, out_ref):
    acc = accP_ref[0] + accP_ref[1]
    den = denP_ref[0:1, :] + denP_ref[1:2, :]
    out_ref[...] = acc / (den + 1e-9) + b2_ref[...]


def _sc1_body(featT_hbm, eler_hbm, eidx_hbm, xagg_hbm,
              feat_v, acc_v, el_v, er_v, den_v, ibuf, sem0, sem1):
    cid = lax.axis_index("c")
    sid = lax.axis_index("s")
    t = sid * 2 + cid
    h = t // 8

    pltpu.sync_copy(featT_hbm.at[pl.ds(4 * t, 4)], feat_v)
    pltpu.sync_copy(eler_hbm.at[pl.ds(h, 1)], el_v)
    pltpu.sync_copy(eler_hbm.at[pl.ds(4 + h, 1)], er_v)

    z = jnp.zeros((16,), jnp.float32)

    @plsc.parallel_loop(0, N // 16, 1, unroll=4)
    def zero_body(j):
        den_v[pl.ds(j * 16, 16)] = z
        for c in range(4):
            acc_v[c, pl.ds(j * 16, 16)] = z

    nchunks = E // CHUNK
    sems = (sem0, sem1)
    pltpu.async_copy(eidx_hbm.at[:, pl.ds(0, CHUNK)], ibuf.at[0], sem0)
    pltpu.async_copy(eidx_hbm.at[:, pl.ds(CHUNK, CHUNK)], ibuf.at[1], sem1)

    z16 = jnp.zeros((16,), jnp.int32)
    cvecs = [jnp.full((16,), c, jnp.int32) for c in range(4)]

    def grp(b):
        @plsc.parallel_loop(0, GPC, 1, unroll=4)
        def inner(g):
            s16 = ibuf[b, 0, pl.ds(g * 16, 16)]
            d16 = ibuf[b, 1, pl.ds(g * 16, 16)]
            a = plsc.load_gather(el_v, [z16, s16])
            r = plsc.load_gather(er_v, [z16, d16])
            e = a + r
            w = jnp.exp(jnp.maximum(e, 0.2 * e))
            plsc.addupdate_scatter(den_v, [d16], w)
            for c in range(4):
                v = plsc.load_gather(feat_v, [cvecs[c], s16])
                plsc.addupdate_scatter(acc_v, [cvecs[c], d16], v * w)

    def chunk_body(k, _):
        for b in (0, 1):
            ch = 2 * k + b
            pltpu.make_async_copy(
                eidx_hbm.at[:, pl.ds(0, CHUNK)], ibuf.at[b], sems[b]).wait()
            grp(b)
            nxt = jnp.minimum((ch + 2) * CHUNK, E - CHUNK)
            pltpu.async_copy(eidx_hbm.at[:, pl.ds(nxt, CHUNK)],
                             ibuf.at[b], sems[b])
        return 0

    lax.fori_loop(0, nchunks // 2, chunk_body, 0)
    pltpu.make_async_copy(eidx_hbm.at[:, pl.ds(0, CHUNK)], ibuf.at[0], sem0).wait()
    pltpu.make_async_copy(eidx_hbm.at[:, pl.ds(0, CHUNK)], ibuf.at[1], sem1).wait()

    one = jnp.full((16,), 1.0, jnp.float32)
    eps = jnp.full((16,), 1e-9, jnp.float32)

    @plsc.parallel_loop(0, N // 16, 1, unroll=4)
    def div_body(j):
        inv = one / (den_v[pl.ds(j * 16, 16)] + eps)
        for c in range(4):
            acc_v[c, pl.ds(j * 16, 16)] = acc_v[c, pl.ds(j * 16, 16)] * inv
    pltpu.sync_copy(acc_v, xagg_hbm.at[pl.ds(4 * t, 4)])


def _sc2_body(featT2_hbm, eler2_hbm, eidx_hbm, accP_hbm, denP_hbm,
              feat_v, acc_v, el_v, er_v, den_v, ibuf, sem0, sem1):
    cid = lax.axis_index("c")
    sid = lax.axis_index("s")
    col = sid
    shard = cid
    base_e = shard * (E // 2)

    pltpu.sync_copy(featT2_hbm.at[pl.ds(col, 1)], feat_v)
    pltpu.sync_copy(eler2_hbm.at[pl.ds(0, 1)], el_v)
    pltpu.sync_copy(eler2_hbm.at[pl.ds(1, 1)], er_v)

    z = jnp.zeros((16,), jnp.float32)

    @plsc.parallel_loop(0, N // 16, 1, unroll=4)
    def zero_body(j):
        den_v[pl.ds(j * 16, 16)] = z
        acc_v[pl.ds(j * 16, 16)] = z

    nchunks = (E // 2) // CHUNK
    sems = (sem0, sem1)
    pltpu.async_copy(eidx_hbm.at[:, pl.ds(base_e, CHUNK)], ibuf.at[0], sem0)
    pltpu.async_copy(eidx_hbm.at[:, pl.ds(base_e + CHUNK, CHUNK)], ibuf.at[1], sem1)

    is_den_tile = col == 0
    z16 = jnp.zeros((16,), jnp.int32)

    def grp(b):
        @plsc.parallel_loop(0, GPC, 1, unroll=4)
        def inner(g):
            s16 = ibuf[b, 0, pl.ds(g * 16, 16)]
            d16 = ibuf[b, 1, pl.ds(g * 16, 16)]
            a = plsc.load_gather(el_v, [z16, s16])
            r = plsc.load_gather(er_v, [z16, d16])
            e = a + r
            w = jnp.exp(jnp.maximum(e, 0.2 * e))

            @pl.when(is_den_tile)
            def _():
                plsc.addupdate_scatter(den_v, [d16], w)

            v = plsc.load_gather(feat_v, [z16, s16])
            plsc.addupdate_scatter(acc_v, [d16], v * w)

    def chunk_body(k, _):
        for b in (0, 1):
            ch = 2 * k + b
            pltpu.make_async_copy(
                eidx_hbm.at[:, pl.ds(0, CHUNK)], ibuf.at[b], sems[b]).wait()
            grp(b)
            nxt = base_e + jnp.minimum((ch + 2) * CHUNK, E // 2 - CHUNK)
            pltpu.async_copy(eidx_hbm.at[:, pl.ds(nxt, CHUNK)],
                             ibuf.at[b], sems[b])
        return 0

    lax.fori_loop(0, nchunks // 2, chunk_body, 0)
    pltpu.make_async_copy(eidx_hbm.at[:, pl.ds(0, CHUNK)], ibuf.at[0], sem0).wait()
    pltpu.make_async_copy(eidx_hbm.at[:, pl.ds(0, CHUNK)], ibuf.at[1], sem1).wait()

    w2id = shard * 16 + col
    pltpu.sync_copy(acc_v, accP_hbm.at[w2id])

    @pl.when(is_den_tile)
    def _():
        pltpu.sync_copy(den_v, denP_hbm.at[shard])


_sc1 = functools.partial(
    pl.kernel,
    out_type=jax.ShapeDtypeStruct((IN_FEATS, N), jnp.float32),
    mesh=_SC_MESH,
    compiler_params=_SC_PARAMS,
    scratch_types=[
        pltpu.VMEM((4, N), jnp.float32),
        pltpu.VMEM((4, N), jnp.float32),
        pltpu.VMEM((1, N), jnp.float32),
        pltpu.VMEM((1, N), jnp.float32),
        pltpu.VMEM((N,), jnp.float32),
        pltpu.VMEM((2, 2, CHUNK), jnp.int32),
        pltpu.SemaphoreType.DMA,
        pltpu.SemaphoreType.DMA,
    ],
)(_sc1_body)

_sc2 = functools.partial(
    pl.kernel,
    out_type=(jax.ShapeDtypeStruct((32, N), jnp.float32),
              jax.ShapeDtypeStruct((2, N), jnp.float32)),
    mesh=_SC_MESH,
    compiler_params=_SC_PARAMS,
    scratch_types=[
        pltpu.VMEM((1, N), jnp.float32),
        pltpu.VMEM((N,), jnp.float32),
        pltpu.VMEM((1, N), jnp.float32),
        pltpu.VMEM((1, N), jnp.float32),
        pltpu.VMEM((N,), jnp.float32),
        pltpu.VMEM((2, 2, CHUNK), jnp.int32),
        pltpu.SemaphoreType.DMA,
        pltpu.SemaphoreType.DMA,
    ],
)(_sc2_body)

_tc1 = pl.pallas_call(
    _tc1_body,
    out_shape=(jax.ShapeDtypeStruct((IN_FEATS, N), jnp.float32),
               jax.ShapeDtypeStruct((2 * HEADS, N), jnp.float32)),
)

_tc2 = pl.pallas_call(
    _tc2_body,
    out_shape=(jax.ShapeDtypeStruct((NUM_CLASSES, N), jnp.float32),
               jax.ShapeDtypeStruct((8, N), jnp.float32)),
)

_tc3 = pl.pallas_call(
    _tc3_body,
    out_shape=jax.ShapeDtypeStruct((NUM_CLASSES, N), jnp.float32),
)


def kernel(features, edge_index, W1, al1, ar1, b1, W2, al2, ar2, b2):
    eidx = edge_index.astype(jnp.int32)

    a1 = jnp.zeros((2 * HEADS, HEADS * HIDDEN), jnp.float32)
    for hh in range(HEADS):
        a1 = a1.at[hh, hh * HIDDEN:(hh + 1) * HIDDEN].set(al1[hh])
        a1 = a1.at[HEADS + hh, hh * HIDDEN:(hh + 1) * HIDDEN].set(ar1[hh])
    a2 = jnp.zeros((8, NUM_CLASSES), jnp.float32)
    a2 = a2.at[0].set(al2[0])
    a2 = a2.at[1].set(ar2[0])

    featT1, eler1 = _tc1(features, W1, a1)
    xaggT = _sc1(featT1, eler1, eidx)
    featT2, eler2 = _tc2(xaggT, b1.reshape(IN_FEATS, 1), W2, a2)
    accP, denP = _sc2(featT2, eler2, eidx)
    accP = accP.reshape(2, NUM_CLASSES, N)
    outT = _tc3(accP, denP, b2.reshape(NUM_CLASSES, 1))
    return outT.T
